# baseline (speedup 1.0000x reference)
import functools

import jax
import jax.numpy as jnp
from jax import lax
from jax.experimental import pallas as pl
from jax.experimental.pallas import tpu as pltpu
from jax.experimental.pallas import tpu_sc as plsc

N = 10000
E = 320000
D = 128
WEXT = 144

NCORES = 2
NSUB = 16
NW = NCORES * NSUB
CHUNK = 128
CH = 80
EPW = CH * CHUNK
GRP = 2
N_ACC = 10112
ROWS_PER_TILE = N_ACC // NSUB


def _make_sc_agg(width, nbuf):
  mesh = plsc.VectorSubcoreMesh(core_axis_name="c", subcore_axis_name="s")

  @functools.partial(
      pl.kernel,
      out_type=jax.ShapeDtypeStruct((NCORES, N_ACC, width), jnp.float32),
      mesh=mesh,
      compiler_params=pltpu.CompilerParams(use_tc_tiling_on_sc=False),
      scratch_types=[
          pltpu.VMEM((GRP, CHUNK), jnp.int32),
          pltpu.VMEM((GRP, CHUNK), jnp.int32),
          pltpu.VMEM((GRP, CHUNK), jnp.int32),
          pltpu.VMEM((GRP, CHUNK), jnp.int32),
          pltpu.VMEM((GRP, CHUNK, width), jnp.float32),
          pltpu.VMEM_SHARED((N_ACC, width), jnp.float32),
      ] + [pltpu.SemaphoreType.DMA] * (2 * GRP + 2),
  )
  def sc_agg(h_hbm, src_hbm, dst_hbm, z_hbm, out_hbm, srcb0, srcb1, dstb0,
             dstb1, msgs, acc, *sems):
    gsem = sems[:GRP]
    ssem = sems[GRP:2 * GRP]
    isem = sems[2 * GRP:]
    srcb = (srcb0, srcb1)
    dstb = (dstb0, dstb1)
    c = lax.axis_index("c")
    s = lax.axis_index("s")
    wid = s * NCORES + c

    def fire_idx(p, c0):
      pltpu.async_copy(src_hbm.at[wid, pl.ds(c0, GRP)], srcb[p], isem[p])
      pltpu.async_copy(dst_hbm.at[wid, pl.ds(c0, GRP)], dstb[p], isem[p])

    def drain_idx(p):
      pltpu.make_async_copy(src_hbm.at[wid, pl.ds(0, GRP)], srcb[p],
                            isem[p]).wait()
      pltpu.make_async_copy(dst_hbm.at[wid, pl.ds(0, GRP)], dstb[p],
                            isem[p]).wait()

    def wait_gather(b):
      pltpu.make_async_copy(h_hbm.at[srcb0.at[0]], msgs.at[b],
                            gsem[b]).wait()

    def scatter_group(p):
      scat = []
      for b in range(GRP):
        wait_gather(b)
        scat.append(pltpu.async_copy(msgs.at[b], acc.at[dstb[p].at[b]],
                                     ssem[b], add=True))
      return scat

    def refill_group(scat, p):
      for b in range(GRP):
        scat[b].wait()
        pltpu.async_copy(h_hbm.at[srcb[p].at[b]], msgs.at[b], gsem[b])

    pltpu.sync_copy(z_hbm, acc.at[pl.ds(s * ROWS_PER_TILE, ROWS_PER_TILE)])
    pltpu.sync_copy(src_hbm.at[wid, pl.ds(0, GRP)], srcb0)
    pltpu.sync_copy(dst_hbm.at[wid, pl.ds(0, GRP)], dstb0)
    plsc.subcore_barrier()
    fire_idx(1, GRP)
    for b in range(GRP):
      pltpu.async_copy(h_hbm.at[srcb0.at[b]], msgs.at[b], gsem[b])

    def sg_body(sg, _):
      j0 = 2 * GRP * sg
      drain_idx(1)
      refill_group(scatter_group(0), 1)
      fire_idx(0, j0 + 2 * GRP)
      scat = scatter_group(1)
      drain_idx(0)
      refill_group(scat, 0)
      fire_idx(1, j0 + 3 * GRP)
      return ()

    lax.fori_loop(0, CH // (2 * GRP) - 1, sg_body, ())

    drain_idx(1)
    refill_group(scatter_group(0), 1)
    for d in scatter_group(1):
      d.wait()
    plsc.subcore_barrier()

    r0 = s * ROWS_PER_TILE
    pltpu.sync_copy(acc.at[pl.ds(r0, ROWS_PER_TILE)],
                    out_hbm.at[c, pl.ds(r0, ROWS_PER_TILE)])

  return sc_agg


_sc_agg_ext = _make_sc_agg(WEXT, 2)
_sc_agg_d = _make_sc_agg(D, 2)

_TC_R = 400


def _tc1_body(h_ref, p_ref, ws_ref, wn_ref, b_ref, g_ref, bb_ref,
              o_ref, d_ref):
  p = p_ref[...]
  ssum = p[0] + p[1]
  agg = ssum[:, :D]
  dinv = 1.0 / jnp.maximum(ssum[:, D], 1.0)
  agg = agg * dinv[:, None]
  out = (jnp.dot(h_ref[...], ws_ref[...], preferred_element_type=jnp.float32)
         + jnp.dot(agg, wn_ref[...], preferred_element_type=jnp.float32)
         + b_ref[...])
  mu = jnp.mean(out, axis=-1, keepdims=True)
  var = jnp.mean((out - mu) * (out - mu), axis=-1, keepdims=True)
  y = (out - mu) * lax.rsqrt(var + 1e-5) * g_ref[...] + bb_ref[...]
  o_ref[...] = jnp.maximum(y, 0.0)
  d_ref[...] = dinv[:, None]


def _tc2_body(h_ref, p_ref, d_ref, ws_ref, wn_ref, b_ref, o_ref):
  p = p_ref[...]
  agg = (p[0] + p[1]) * d_ref[...]
  o_ref[...] = (
      jnp.dot(h_ref[...], ws_ref[...], preferred_element_type=jnp.float32)
      + jnp.dot(agg, wn_ref[...], preferred_element_type=jnp.float32)
      + b_ref[...])


def _full(shape):
  return pl.BlockSpec(shape, lambda i: (0,) * len(shape))


def _tc_layer1(feat, parts, W_self, W_neigh, b, g, beta):
  grid = (N // _TC_R,)
  return pl.pallas_call(
      _tc1_body,
      grid=grid,
      in_specs=[
          pl.BlockSpec((_TC_R, D), lambda i: (i, 0)),
          pl.BlockSpec((NCORES, _TC_R, WEXT), lambda i: (0, i, 0)),
          _full((D, D)),
          _full((D, D)),
          _full((1, D)),
          _full((1, D)),
          _full((1, D)),
      ],
      out_specs=[
          pl.BlockSpec((_TC_R, D), lambda i: (i, 0)),
          pl.BlockSpec((_TC_R, 1), lambda i: (i, 0)),
      ],
      out_shape=[
          jax.ShapeDtypeStruct((N, D), jnp.float32),
          jax.ShapeDtypeStruct((N, 1), jnp.float32),
      ],
  )(feat, parts, W_self, W_neigh, b.reshape(1, D), g.reshape(1, D),
    beta.reshape(1, D))


def _tc_layer2(h, parts, dinv, W_self, W_neigh, b):
  grid = (N // _TC_R,)
  return pl.pallas_call(
      _tc2_body,
      grid=grid,
      in_specs=[
          pl.BlockSpec((_TC_R, D), lambda i: (i, 0)),
          pl.BlockSpec((NCORES, _TC_R, D), lambda i: (0, i, 0)),
          pl.BlockSpec((_TC_R, 1), lambda i: (i, 0)),
          _full((D, D)),
          _full((D, D)),
          _full((1, D)),
      ],
      out_specs=pl.BlockSpec((_TC_R, D), lambda i: (i, 0)),
      out_shape=jax.ShapeDtypeStruct((N, D), jnp.float32),
  )(h, parts, dinv, W_self, W_neigh, b.reshape(1, D))


def kernel(feat, edge_index, W_self0, W_neigh0, b0, W_self1, W_neigh1, b1,
           ln_g, ln_b):
  epw_real = E // NW
  pad = EPW - epw_real
  src = jnp.pad(edge_index[0].reshape(NW, epw_real), ((0, 0), (0, pad)),
                constant_values=0).reshape(NW, CH, CHUNK)
  dst = jnp.pad(edge_index[1].reshape(NW, epw_real), ((0, 0), (0, pad)),
                constant_values=N).reshape(NW, CH, CHUNK)

  feat_ext = jnp.concatenate(
      [feat, jnp.ones((N, 1), jnp.float32), jnp.zeros((N, WEXT - D - 1),
                                                      jnp.float32)], axis=1)
  z_ext = jnp.zeros((ROWS_PER_TILE, WEXT), jnp.float32)
  z_d = jnp.zeros((ROWS_PER_TILE, D), jnp.float32)

  parts1 = _sc_agg_ext(feat_ext, src, dst, z_ext)
  h1, dinv = _tc_layer1(feat, parts1, W_self0, W_neigh0, b0, ln_g, ln_b)
  parts2 = _sc_agg_d(h1, src, dst, z_d)
  return _tc_layer2(h1, parts2, dinv, W_self1, W_neigh1, b1)


# speedup vs baseline: 1.0780x; 1.0780x over previous
import functools

import jax
import jax.numpy as jnp
from jax import lax
from jax.experimental import pallas as pl
from jax.experimental.pallas import tpu as pltpu
from jax.experimental.pallas import tpu_sc as plsc

N = 10000
E = 320000
D = 128
WEXT = 144

NCORES = 2
NSUB = 16
NW = NCORES * NSUB
CHUNK = 128
CH = 80
EPW = CH * CHUNK
GRP = 2
N_ACC = 10112
ROWS_PER_TILE = N_ACC // NSUB


def _make_sc_agg(width, nbuf):
  mesh = plsc.VectorSubcoreMesh(core_axis_name="c", subcore_axis_name="s")

  @functools.partial(
      pl.kernel,
      out_type=jax.ShapeDtypeStruct((NCORES, N_ACC, width), jnp.float32),
      mesh=mesh,
      compiler_params=pltpu.CompilerParams(use_tc_tiling_on_sc=False),
      scratch_types=[
          pltpu.VMEM((GRP, CHUNK), jnp.int32),
          pltpu.VMEM((GRP, CHUNK), jnp.int32),
          pltpu.VMEM((GRP, CHUNK), jnp.int32),
          pltpu.VMEM((GRP, CHUNK), jnp.int32),
          pltpu.VMEM((GRP, CHUNK, width), jnp.float32),
          pltpu.VMEM_SHARED((N_ACC, width), jnp.float32),
      ] + [pltpu.SemaphoreType.DMA] * (2 * GRP + 2),
  )
  def sc_agg(h_hbm, src_hbm, dst_hbm, z_hbm, out_hbm, srcb0, srcb1, dstb0,
             dstb1, msgs, acc, *sems):
    gsem = sems[:GRP]
    ssem = sems[GRP:2 * GRP]
    isem = sems[2 * GRP:]
    srcb = (srcb0, srcb1)
    dstb = (dstb0, dstb1)
    c = lax.axis_index("c")
    s = lax.axis_index("s")
    wid = s * NCORES + c

    def fire_idx(p, c0):
      pltpu.async_copy(src_hbm.at[wid, pl.ds(c0, GRP)], srcb[p], isem[p])
      pltpu.async_copy(dst_hbm.at[wid, pl.ds(c0, GRP)], dstb[p], isem[p])

    def drain_idx(p):
      pltpu.make_async_copy(src_hbm.at[wid, pl.ds(0, GRP)], srcb[p],
                            isem[p]).wait()
      pltpu.make_async_copy(dst_hbm.at[wid, pl.ds(0, GRP)], dstb[p],
                            isem[p]).wait()

    def wait_gather(b):
      pltpu.make_async_copy(h_hbm.at[srcb0.at[0]], msgs.at[b],
                            gsem[b]).wait()

    def process_group(p, refill, drain_p=None):
      for b in range(GRP):
        wait_gather(b)
        pltpu.sync_copy(msgs.at[b], acc.at[dstb[p].at[b]], add=True)
        if b == 0 and drain_p is not None:
          drain_idx(drain_p)
        if refill:
          pltpu.async_copy(h_hbm.at[srcb[1 - p].at[b]], msgs.at[b], gsem[b])

    pltpu.sync_copy(z_hbm, acc.at[pl.ds(s * ROWS_PER_TILE, ROWS_PER_TILE)])
    pltpu.sync_copy(src_hbm.at[wid, pl.ds(0, GRP)], srcb0)
    pltpu.sync_copy(dst_hbm.at[wid, pl.ds(0, GRP)], dstb0)
    plsc.subcore_barrier()
    fire_idx(1, GRP)
    for b in range(GRP):
      pltpu.async_copy(h_hbm.at[srcb0.at[b]], msgs.at[b], gsem[b])

    def sg_body(sg, _):
      j0 = 2 * GRP * sg
      drain_idx(1)
      process_group(0, True)
      fire_idx(0, j0 + 2 * GRP)
      process_group(1, True, drain_p=0)
      fire_idx(1, j0 + 3 * GRP)
      return ()

    lax.fori_loop(0, CH // (2 * GRP) - 1, sg_body, ())

    drain_idx(1)
    process_group(0, True)
    process_group(1, False)
    plsc.subcore_barrier()

    r0 = s * ROWS_PER_TILE
    pltpu.sync_copy(acc.at[pl.ds(r0, ROWS_PER_TILE)],
                    out_hbm.at[c, pl.ds(r0, ROWS_PER_TILE)])

  return sc_agg


_sc_agg_ext = _make_sc_agg(WEXT, 2)
_sc_agg_d = _make_sc_agg(D, 2)

_TC_R = 400


def _tc1_body(h_ref, p_ref, ws_ref, wn_ref, b_ref, g_ref, bb_ref,
              o_ref, d_ref):
  p = p_ref[...]
  ssum = p[0] + p[1]
  agg = ssum[:, :D]
  dinv = 1.0 / jnp.maximum(ssum[:, D], 1.0)
  agg = agg * dinv[:, None]
  out = (jnp.dot(h_ref[...], ws_ref[...], preferred_element_type=jnp.float32)
         + jnp.dot(agg, wn_ref[...], preferred_element_type=jnp.float32)
         + b_ref[...])
  mu = jnp.mean(out, axis=-1, keepdims=True)
  var = jnp.mean((out - mu) * (out - mu), axis=-1, keepdims=True)
  y = (out - mu) * lax.rsqrt(var + 1e-5) * g_ref[...] + bb_ref[...]
  o_ref[...] = jnp.maximum(y, 0.0)
  d_ref[...] = dinv[:, None]


def _tc2_body(h_ref, p_ref, d_ref, ws_ref, wn_ref, b_ref, o_ref):
  p = p_ref[...]
  agg = (p[0] + p[1]) * d_ref[...]
  o_ref[...] = (
      jnp.dot(h_ref[...], ws_ref[...], preferred_element_type=jnp.float32)
      + jnp.dot(agg, wn_ref[...], preferred_element_type=jnp.float32)
      + b_ref[...])


def _full(shape):
  return pl.BlockSpec(shape, lambda i: (0,) * len(shape))


def _tc_layer1(feat, parts, W_self, W_neigh, b, g, beta):
  grid = (N // _TC_R,)
  return pl.pallas_call(
      _tc1_body,
      grid=grid,
      in_specs=[
          pl.BlockSpec((_TC_R, D), lambda i: (i, 0)),
          pl.BlockSpec((NCORES, _TC_R, WEXT), lambda i: (0, i, 0)),
          _full((D, D)),
          _full((D, D)),
          _full((1, D)),
          _full((1, D)),
          _full((1, D)),
      ],
      out_specs=[
          pl.BlockSpec((_TC_R, D), lambda i: (i, 0)),
          pl.BlockSpec((_TC_R, 1), lambda i: (i, 0)),
      ],
      out_shape=[
          jax.ShapeDtypeStruct((N, D), jnp.float32),
          jax.ShapeDtypeStruct((N, 1), jnp.float32),
      ],
  )(feat, parts, W_self, W_neigh, b.reshape(1, D), g.reshape(1, D),
    beta.reshape(1, D))


def _tc_layer2(h, parts, dinv, W_self, W_neigh, b):
  grid = (N // _TC_R,)
  return pl.pallas_call(
      _tc2_body,
      grid=grid,
      in_specs=[
          pl.BlockSpec((_TC_R, D), lambda i: (i, 0)),
          pl.BlockSpec((NCORES, _TC_R, D), lambda i: (0, i, 0)),
          pl.BlockSpec((_TC_R, 1), lambda i: (i, 0)),
          _full((D, D)),
          _full((D, D)),
          _full((1, D)),
      ],
      out_specs=pl.BlockSpec((_TC_R, D), lambda i: (i, 0)),
      out_shape=jax.ShapeDtypeStruct((N, D), jnp.float32),
  )(h, parts, dinv, W_self, W_neigh, b.reshape(1, D))


def kernel(feat, edge_index, W_self0, W_neigh0, b0, W_self1, W_neigh1, b1,
           ln_g, ln_b):
  epw_real = E // NW
  pad = EPW - epw_real
  src = jnp.pad(edge_index[0].reshape(NW, epw_real), ((0, 0), (0, pad)),
                constant_values=0).reshape(NW, CH, CHUNK)
  dst = jnp.pad(edge_index[1].reshape(NW, epw_real), ((0, 0), (0, pad)),
                constant_values=N).reshape(NW, CH, CHUNK)

  feat_ext = jnp.concatenate(
      [feat, jnp.ones((N, 1), jnp.float32), jnp.zeros((N, WEXT - D - 1),
                                                      jnp.float32)], axis=1)
  z_ext = jnp.zeros((ROWS_PER_TILE, WEXT), jnp.float32)
  z_d = jnp.zeros((ROWS_PER_TILE, D), jnp.float32)

  parts1 = _sc_agg_ext(feat_ext, src, dst, z_ext)
  h1, dinv = _tc_layer1(feat, parts1, W_self0, W_neigh0, b0, ln_g, ln_b)
  parts2 = _sc_agg_d(h1, src, dst, z_d)
  return _tc_layer2(h1, parts2, dinv, W_self1, W_neigh1, b1)


# speedup vs baseline: 1.0848x; 1.0063x over previous
import functools

import jax
import jax.numpy as jnp
from jax import lax
from jax.experimental import pallas as pl
from jax.experimental.pallas import tpu as pltpu
from jax.experimental.pallas import tpu_sc as plsc

N = 10000
E = 320000
D = 128
WEXT = 144

NCORES = 2
NSUB = 16
NW = NCORES * NSUB
CHUNK = 64
CH = 160
EPW = CH * CHUNK
NBUF = 4
SLAG = 2
NG = CH // NBUF
NSG = NG // 4
N_ACC = 10112
ROWS_PER_TILE = N_ACC // NSUB


def _make_sc_agg(width):
  mesh = plsc.VectorSubcoreMesh(core_axis_name="c", subcore_axis_name="s")

  @functools.partial(
      pl.kernel,
      out_type=jax.ShapeDtypeStruct((NCORES, N_ACC, width), jnp.float32),
      mesh=mesh,
      compiler_params=pltpu.CompilerParams(use_tc_tiling_on_sc=False),
      scratch_types=[
          pltpu.VMEM((4, NBUF, CHUNK), jnp.int32),
          pltpu.VMEM((4, NBUF, CHUNK), jnp.int32),
          pltpu.VMEM((NBUF, CHUNK, width), jnp.float32),
          pltpu.VMEM_SHARED((N_ACC, width), jnp.float32),
      ] + [pltpu.SemaphoreType.DMA] * (3 * NBUF),
  )
  def sc_agg(h_hbm, src_hbm, dst_hbm, z_hbm, out_hbm, srcb, dstb, msgs, acc,
             *sems):
    gsem = sems[:NBUF]
    ssem = sems[NBUF:2 * NBUF]
    isem = sems[2 * NBUF:]
    c = lax.axis_index("c")
    s = lax.axis_index("s")
    wid = s * NCORES + c

    def fire_idx(slot, grp):
      pltpu.async_copy(src_hbm.at[wid, pl.ds(grp * NBUF, NBUF)],
                       srcb.at[slot], isem[slot])
      pltpu.async_copy(dst_hbm.at[wid, pl.ds(grp * NBUF, NBUF)],
                       dstb.at[slot], isem[slot])

    def wait_idx(slot):
      pltpu.make_async_copy(src_hbm.at[wid, pl.ds(0, NBUF)], srcb.at[slot],
                            isem[slot]).wait()
      pltpu.make_async_copy(dst_hbm.at[wid, pl.ds(0, NBUF)], dstb.at[slot],
                            isem[slot]).wait()

    def fire_gather(islot, row, b):
      pltpu.async_copy(h_hbm.at[srcb.at[islot, row]], msgs.at[b], gsem[b])

    def wait_gather(b):
      pltpu.make_async_copy(h_hbm.at[srcb.at[0, 0]], msgs.at[b],
                            gsem[b]).wait()

    def fire_scatter(islot, row, b):
      pltpu.async_copy(msgs.at[b], acc.at[dstb.at[islot, row]], ssem[b],
                       add=True)

    def wait_scatter(b):
      pltpu.make_async_copy(msgs.at[b], acc.at[dstb.at[0, 0]],
                            ssem[b]).wait()

    def group(kg, kslot, idx_wait, idx_fire, head=False, tail=False):
      if idx_wait:
        wait_idx((kslot + 1) % 4)
      if idx_fire:
        fire_idx((kslot + 2) % 4, kg + 2)
      for b in range(NBUF):
        if head and b < SLAG:
          wait_gather(b)
          fire_scatter(kslot, b, b)
          continue
        wait_scatter((b + SLAG) % NBUF)
        if not tail:
          if b < NBUF - SLAG:
            fire_gather(kslot, b + SLAG, (b + SLAG) % NBUF)
          else:
            fire_gather((kslot + 1) % 4, b - (NBUF - SLAG), (b + SLAG) % NBUF)
        elif b < SLAG:
          fire_gather(kslot, b + SLAG, (b + SLAG) % NBUF)
        wait_gather(b)
        fire_scatter(kslot, b, b)
      if tail:
        for b in range(SLAG):
          wait_scatter((b + SLAG) % NBUF)

    pltpu.sync_copy(z_hbm, acc.at[pl.ds(s * ROWS_PER_TILE, ROWS_PER_TILE)])
    pltpu.sync_copy(src_hbm.at[wid, pl.ds(0, NBUF)], srcb.at[0])
    pltpu.sync_copy(dst_hbm.at[wid, pl.ds(0, NBUF)], dstb.at[0])
    fire_idx(1, 1)
    plsc.subcore_barrier()
    for b in range(NBUF):
      fire_gather(0, b, b)

    group(0, 0, idx_wait=True, idx_fire=True, head=True)
    for kg in range(1, 4):
      group(kg, kg, idx_wait=True, idx_fire=True)

    def body(sg, _):
      for j in range(4):
        group(4 * sg + j, j, idx_wait=True, idx_fire=True)
      return ()

    lax.fori_loop(1, NSG - 1, body, ())

    for kg in range(NG - 4, NG - 2):
      group(kg, kg % 4, idx_wait=True, idx_fire=True)
    group(NG - 2, (NG - 2) % 4, idx_wait=True, idx_fire=False)
    group(NG - 1, (NG - 1) % 4, idx_wait=False, idx_fire=False, tail=True)
    plsc.subcore_barrier()

    r0 = s * ROWS_PER_TILE
    pltpu.sync_copy(acc.at[pl.ds(r0, ROWS_PER_TILE)],
                    out_hbm.at[c, pl.ds(r0, ROWS_PER_TILE)])

  return sc_agg


_sc_agg_ext = _make_sc_agg(WEXT)
_sc_agg_d = _make_sc_agg(D)

_TC_R = 400


def _tc_self_body(h_ref, ws_ref, b_ref, o_ref):
  o_ref[...] = (
      jnp.dot(h_ref[...], ws_ref[...], preferred_element_type=jnp.float32)
      + b_ref[...])


def _tc1_body(sp_ref, p_ref, wn_ref, g_ref, bb_ref, o_ref, d_ref):
  p = p_ref[...]
  ssum = p[0] + p[1]
  agg = ssum[:, :D]
  dinv = 1.0 / jnp.maximum(ssum[:, D], 1.0)
  agg = agg * dinv[:, None]
  out = (sp_ref[...]
         + jnp.dot(agg, wn_ref[...], preferred_element_type=jnp.float32))
  mu = jnp.mean(out, axis=-1, keepdims=True)
  var = jnp.mean((out - mu) * (out - mu), axis=-1, keepdims=True)
  y = (out - mu) * lax.rsqrt(var + 1e-5) * g_ref[...] + bb_ref[...]
  o_ref[...] = jnp.maximum(y, 0.0)
  d_ref[...] = dinv[:, None]


def _tc2_body(sp_ref, p_ref, d_ref, wn_ref, o_ref):
  p = p_ref[...]
  agg = (p[0] + p[1]) * d_ref[...]
  o_ref[...] = (
      sp_ref[...]
      + jnp.dot(agg, wn_ref[...], preferred_element_type=jnp.float32))


def _full(shape):
  return pl.BlockSpec(shape, lambda i: (0,) * len(shape))


def _tc_self(h, W_self, b):
  return pl.pallas_call(
      _tc_self_body,
      grid=(N // _TC_R,),
      in_specs=[
          pl.BlockSpec((_TC_R, D), lambda i: (i, 0)),
          _full((D, D)),
          _full((1, D)),
      ],
      out_specs=pl.BlockSpec((_TC_R, D), lambda i: (i, 0)),
      out_shape=jax.ShapeDtypeStruct((N, D), jnp.float32),
  )(h, W_self, b.reshape(1, D))


def _tc_layer1(sp, parts, W_neigh, g, beta):
  grid = (N // _TC_R,)
  return pl.pallas_call(
      _tc1_body,
      grid=grid,
      in_specs=[
          pl.BlockSpec((_TC_R, D), lambda i: (i, 0)),
          pl.BlockSpec((NCORES, _TC_R, WEXT), lambda i: (0, i, 0)),
          _full((D, D)),
          _full((1, D)),
          _full((1, D)),
      ],
      out_specs=[
          pl.BlockSpec((_TC_R, D), lambda i: (i, 0)),
          pl.BlockSpec((_TC_R, 1), lambda i: (i, 0)),
      ],
      out_shape=[
          jax.ShapeDtypeStruct((N, D), jnp.float32),
          jax.ShapeDtypeStruct((N, 1), jnp.float32),
      ],
  )(sp, parts, W_neigh, g.reshape(1, D), beta.reshape(1, D))


def _tc_layer2(sp, parts, dinv, W_neigh):
  grid = (N // _TC_R,)
  return pl.pallas_call(
      _tc2_body,
      grid=grid,
      in_specs=[
          pl.BlockSpec((_TC_R, D), lambda i: (i, 0)),
          pl.BlockSpec((NCORES, _TC_R, D), lambda i: (0, i, 0)),
          pl.BlockSpec((_TC_R, 1), lambda i: (i, 0)),
          _full((D, D)),
      ],
      out_specs=pl.BlockSpec((_TC_R, D), lambda i: (i, 0)),
      out_shape=jax.ShapeDtypeStruct((N, D), jnp.float32),
  )(sp, parts, dinv, W_neigh)


def kernel(feat, edge_index, W_self0, W_neigh0, b0, W_self1, W_neigh1, b1,
           ln_g, ln_b):
  epw_real = E // NW
  pad = EPW - epw_real
  src = jnp.pad(edge_index[0].reshape(NW, epw_real), ((0, 0), (0, pad)),
                constant_values=0).reshape(NW, CH, CHUNK)
  dst = jnp.pad(edge_index[1].reshape(NW, epw_real), ((0, 0), (0, pad)),
                constant_values=N).reshape(NW, CH, CHUNK)

  feat_ext = jnp.concatenate(
      [feat, jnp.ones((N, 1), jnp.float32), jnp.zeros((N, WEXT - D - 1),
                                                      jnp.float32)], axis=1)
  z_ext = jnp.zeros((ROWS_PER_TILE, WEXT), jnp.float32)
  z_d = jnp.zeros((ROWS_PER_TILE, D), jnp.float32)

  sp1 = _tc_self(feat, W_self0, b0)
  parts1 = _sc_agg_ext(feat_ext, src, dst, z_ext)
  h1, dinv = _tc_layer1(sp1, parts1, W_neigh0, ln_g, ln_b)
  sp2 = _tc_self(h1, W_self1, b1)
  parts2 = _sc_agg_d(h1, src, dst, z_d)
  return _tc_layer2(sp2, parts2, dinv, W_neigh1)
